# baseline (speedup 1.0000x reference)
import functools

import jax
import jax.numpy as jnp
from jax import lax
from jax.experimental import pallas as pl
from jax.experimental.pallas import tpu as pltpu
from jax.experimental.pallas import tpu_sc as plsc

B = 16384
D = 64
L = 16
NC = 2
NS = 16
NW = NC * NS
PER_W = B // NW
C = 128
NCHUNK = PER_W // C


def _tk_body(h_hbm, t_hbm, r_hbm, ee_hbm, re_hbm, et_hbm, rt_hbm, out_hbm,
             idx_v, rows_v, out_v, sem0, sem1):
    wid = lax.axis_index("s") * NC + lax.axis_index("c")
    base = wid * PER_W
    sems = (sem0, sem1)
    tables = (ee_hbm, ee_hbm, et_hbm, et_hbm, re_hbm, rt_hbm)
    idx_of = (0, 1, 0, 1, 2, 2)
    descs = [None, None]

    def issue(c):
        s = c % 2
        off = base + c * C
        pltpu.sync_copy(h_hbm.at[pl.ds(off, C)], idx_v.at[s, 0])
        pltpu.sync_copy(t_hbm.at[pl.ds(off, C)], idx_v.at[s, 1])
        pltpu.sync_copy(r_hbm.at[pl.ds(off, C)], idx_v.at[s, 2])
        descs[s] = [
            pltpu.async_copy(tables[j].at[idx_v.at[s, idx_of[j]]],
                             rows_v.at[s, j], sems[s])
            for j in range(6)
        ]

    def compute(c):
        s = c % 2
        lane = lax.iota(jnp.int32, L)

        @plsc.parallel_loop(0, C, step=L)
        def _grp(g):
            pv = jnp.zeros((L,), jnp.float32)
            for j in range(L):
                i = g + j
                he = [rows_v[s, 0, i, pl.ds(k * L, L)] for k in range(D // L)]
                te = [rows_v[s, 1, i, pl.ds(k * L, L)] for k in range(D // L)]
                acc1 = jnp.zeros((L,), jnp.float32)
                for k in range(D // L):
                    ht = rows_v[s, 2, i, pl.ds(k * L, L)]
                    tt = rows_v[s, 3, i, pl.ds(k * L, L)]
                    acc1 = acc1 + (he[k] * ht - te[k] * tt)
                sv = jnp.sum(acc1)
                acc2 = jnp.zeros((L,), jnp.float32)
                for k in range(D // L):
                    re_ = rows_v[s, 4, i, pl.ds(k * L, L)]
                    rt_ = rows_v[s, 5, i, pl.ds(k * L, L)]
                    acc2 = acc2 + jnp.abs(he[k] - te[k] + re_ + sv * rt_)
                pv = jnp.where(lane == j, jnp.sum(acc2), pv)
            out_v[pl.ds(c * C + g, L)] = pv

    issue(0)
    for c in range(NCHUNK):
        if c + 1 < NCHUNK:
            issue(c + 1)
        for d in descs[c % 2]:
            d.wait()
        compute(c)
    pltpu.sync_copy(out_v, out_hbm.at[pl.ds(base, PER_W)])


@jax.jit
def _transd_sc(h, t, r, ent_embeddings, rel_embeddings, ent_transfer,
               rel_transfer):
    mesh = plsc.VectorSubcoreMesh(core_axis_name="c", subcore_axis_name="s")
    f = pl.kernel(
        _tk_body,
        out_type=jax.ShapeDtypeStruct((B,), jnp.float32),
        mesh=mesh,
        compiler_params=pltpu.CompilerParams(needs_layout_passes=False,
                                             use_tc_tiling_on_sc=False),
        scratch_types=[
            pltpu.VMEM((2, 3, C), jnp.int32),
            pltpu.VMEM((2, 6, C, D), jnp.float32),
            pltpu.VMEM((PER_W,), jnp.float32),
            pltpu.SemaphoreType.DMA,
            pltpu.SemaphoreType.DMA,
        ],
    )
    return f(h, t, r, ent_embeddings, rel_embeddings, ent_transfer,
             rel_transfer)


def kernel(h, t, r, ent_embeddings, rel_embeddings, ent_transfer,
           rel_transfer):
    out = _transd_sc(h, t, r, ent_embeddings, rel_embeddings, ent_transfer,
                     rel_transfer)
    return out.reshape(B, 1)
